# Initial kernel scaffold; baseline (speedup 1.0000x reference)
import jax
import jax.numpy as jnp
from jax.experimental import pallas as pl


def kernel(x, Wl, bl, Wr, br, W1, b1, W2, b2):
    raise NotImplementedError("write your pallas kernel here")


# speedup vs baseline: 4.3501x; 4.3501x over previous
import functools

import jax
import jax.numpy as jnp
from jax.experimental import pallas as pl
from jax.experimental.pallas import tpu as pltpu

D_MODEL = 1024
EXPERT_SIZE = 4096
N_EXP = 8
N_LANG = 8
S = 2048
F_BLK = 512
N_FBLK = EXPERT_SIZE // F_BLK


def _moe_body(x_ref, w_ref, W1_ref, b1_ref, W2_ref, b2_ref, out_ref, acc_ref):
    e = pl.program_id(0)
    f = pl.program_id(1)

    wcol = w_ref[0, 0][:, None]

    @pl.when((e == 0) & (f == 0))
    def _init():
        acc_ref[...] = jnp.zeros_like(acc_ref)

    @pl.when(f == 0)
    def _bias2():
        acc_ref[...] += wcol * b2_ref[0, 0][None, :]

    x = x_ref[...]
    w1 = W1_ref[0]
    h = jax.lax.dot_general(x, w1, (((1,), (1,)), ((), ())),
                            preferred_element_type=jnp.float32)
    h = h + b1_ref[0, 0, 0][None, :]
    g = 0.5 * h * (1.0 + jax.lax.erf(h * 0.7071067811865476))
    w2 = W2_ref[0]
    part = jax.lax.dot_general(g, w2, (((1,), (1,)), ((), ())),
                               preferred_element_type=jnp.float32)
    acc_ref[...] += wcol * part

    @pl.when((e == N_EXP - 1) & (f == N_FBLK - 1))
    def _fin():
        out_ref[...] = acc_ref[...]


@functools.partial(jax.jit, static_argnums=())
def kernel(x, Wl, bl, Wr, br, W1, b1, W2, b2):
    xf = x.reshape(S, D_MODEL)

    lang_logits = xf @ Wl.T + bl
    lp = jax.nn.softmax(lang_logits, axis=-1)
    ew = lp + jnp.concatenate([lp[:, 1:], jnp.zeros((S, 1), lp.dtype)], axis=1)
    rl = xf @ Wr.T + br + 0.1 * ew
    rp = jax.nn.softmax(rl, axis=-1)
    p2, i2 = jax.lax.top_k(rp, 2)
    p2 = p2 / p2.sum(axis=-1, keepdims=True)
    w = (p2[:, 0:1] * jax.nn.one_hot(i2[:, 0], N_EXP, dtype=xf.dtype)
         + p2[:, 1:2] * jax.nn.one_hot(i2[:, 1], N_EXP, dtype=xf.dtype))
    wT = w.T.reshape(N_EXP, 1, S)

    b1r = b1.reshape(N_EXP, N_FBLK, 1, F_BLK)
    b2r = b2.reshape(N_EXP, 1, D_MODEL)

    out = pl.pallas_call(
        _moe_body,
        grid=(N_EXP, N_FBLK),
        in_specs=[
            pl.BlockSpec((S, D_MODEL), lambda e, f: (0, 0)),
            pl.BlockSpec((1, 1, S), lambda e, f: (e, 0, 0)),
            pl.BlockSpec((1, F_BLK, D_MODEL), lambda e, f: (e, f, 0)),
            pl.BlockSpec((1, 1, 1, F_BLK), lambda e, f: (e, f, 0, 0)),
            pl.BlockSpec((1, D_MODEL, F_BLK), lambda e, f: (e, 0, f)),
            pl.BlockSpec((1, 1, D_MODEL), lambda e, f: (e, 0, 0)),
        ],
        out_specs=pl.BlockSpec((S, D_MODEL), lambda e, f: (0, 0)),
        out_shape=jax.ShapeDtypeStruct((S, D_MODEL), jnp.float32),
        scratch_shapes=[pltpu.VMEM((S, D_MODEL), jnp.float32)],
    )(xf, wT, W1, b1r, W2, b2r)

    return out.reshape(x.shape)
